# baseline (speedup 1.0000x reference)
import dataclasses
import functools

import jax
import jax.numpy as jnp
from jax import lax
from jax.experimental import pallas as pl
from jax.experimental.pallas import tpu as pltpu
from jax.experimental.pallas import tpu_sc as plsc

B = 16384
N_NUM = 13
F = 26
V = 100000
K = 16
NC = 2
NS = 16
NW = NC * NS
ROWS_W = B // NW
RC = 4
NCHUNK = ROWS_W // RC
FP = 32


def _sc_gather(vflat, lflat, idxv, idxl):
    mesh = plsc.VectorSubcoreMesh(core_axis_name="c", subcore_axis_name="s")
    cp = pltpu.CompilerParams()
    if "use_tc_tiling_on_sc" in pltpu.CompilerParams.__dataclass_fields__:
        cp = dataclasses.replace(cp, use_tc_tiling_on_sc=False)

    @functools.partial(
        pl.kernel,
        out_type=jax.ShapeDtypeStruct((B, 3 * K), jnp.float32),
        mesh=mesh,
        compiler_params=cp,
        scratch_types=[
            pltpu.VMEM((ROWS_W * F,), jnp.int32),
            pltpu.VMEM((ROWS_W * FP,), jnp.int32),
            pltpu.VMEM((RC * F, K), jnp.float32),
            pltpu.VMEM((RC * FP,), jnp.float32),
            pltpu.VMEM((ROWS_W, 3 * K), jnp.float32),
            pltpu.SemaphoreType.DMA,
            pltpu.SemaphoreType.DMA,
        ],
    )
    def k(vflat_hbm, lflat_hbm, idxv_hbm, idxl_hbm, out_hbm,
          idxv_v, idxl_v, vrows, lrows, outbuf, sem_v, sem_l):
        wid = lax.axis_index("s") * NC + lax.axis_index("c")
        base = wid * ROWS_W
        pltpu.sync_copy(idxv_hbm.at[pl.ds(base * F, ROWS_W * F)], idxv_v)
        pltpu.sync_copy(idxl_hbm.at[pl.ds(base * FP, ROWS_W * FP)], idxl_v)

        @pl.loop(0, NCHUNK)
        def _(c):
            cv = pltpu.async_copy(
                vflat_hbm.at[idxv_v.at[pl.ds(c * (RC * F), RC * F)]],
                vrows, sem_v)
            cl = pltpu.async_copy(
                lflat_hbm.at[idxl_v.at[pl.ds(c * (RC * FP), RC * FP)]],
                lrows, sem_l)
            cv.wait()
            cl.wait()
            for r in range(RC):
                acc = vrows[r * F]
                acc2 = acc * acc
                for j in range(1, F):
                    e = vrows[r * F + j]
                    acc = acc + e
                    acc2 = acc2 + e * e
                lp = lrows[pl.ds(r * FP, K)] + lrows[pl.ds(r * FP + K, K)]
                row = c * RC + r
                outbuf[row, pl.ds(0, K)] = acc
                outbuf[row, pl.ds(K, K)] = acc2
                outbuf[row, pl.ds(2 * K, K)] = lp

        pltpu.sync_copy(outbuf, out_hbm.at[pl.ds(base, ROWS_W)])

    return k(vflat, lflat, idxv, idxl)


def _combine(scout, x_num, v_num, w_row, const):
    BLK = 512

    def body(sc_ref, x_ref, vn_ref, w_ref, c_ref, o_ref):
        sc = sc_ref[...]
        x = x_ref[...]
        vn = vn_ref[...]
        sv = sc[:, 0:K] + jnp.dot(x, vn, preferred_element_type=jnp.float32)
        sq = sc[:, K:2 * K] + jnp.dot(x * x, vn * vn,
                                      preferred_element_type=jnp.float32)
        lin = (jnp.sum(sc[:, 2 * K:3 * K], axis=1, keepdims=True)
               + jnp.sum(x * w_ref[...], axis=1, keepdims=True)
               + c_ref[0, 0])
        o_ref[...] = lin + 0.5 * jnp.sum(sv * sv - sq, axis=1, keepdims=True)

    return pl.pallas_call(
        body,
        grid=(B // BLK,),
        in_specs=[
            pl.BlockSpec((BLK, 3 * K), lambda i: (i, 0)),
            pl.BlockSpec((BLK, N_NUM), lambda i: (i, 0)),
            pl.BlockSpec((N_NUM, K), lambda i: (0, 0)),
            pl.BlockSpec((1, N_NUM), lambda i: (0, 0)),
            pl.BlockSpec((1, 1), lambda i: (0, 0)),
        ],
        out_specs=pl.BlockSpec((BLK, 1), lambda i: (i, 0)),
        out_shape=jax.ShapeDtypeStruct((B, 1), jnp.float32),
    )(scout, x_num, v_num, w_row, const)


def kernel(x_num, x_cat, bias, W_num, lin_cat, v_num, v_cat):
    xc = x_cat.astype(jnp.int32)
    offs = jnp.arange(F, dtype=jnp.int32) * V
    idxv2 = xc + offs[None, :]
    idxv = idxv2.reshape(-1)
    idxl = jnp.concatenate(
        [idxv2, jnp.zeros((B, FP - F), jnp.int32)], axis=1).reshape(-1)
    vflat = v_cat.reshape(F * V, K)
    lflat = lin_cat.reshape(F * V)
    scout = _sc_gather(vflat, lflat, idxv, idxl)
    const = (bias[0] - (FP - F) * lin_cat[0, 0, 0]).reshape(1, 1)
    return _combine(scout, x_num, v_num, W_num, const)
